# Initial kernel scaffold; baseline (speedup 1.0000x reference)
import jax
import jax.numpy as jnp
from jax.experimental import pallas as pl


def kernel(H, E, ht, W_fwd, b_fwd, W_back, b_back, gamma, beta):
    raise NotImplementedError("write your pallas kernel here")


# speedup vs baseline: 3.0133x; 3.0133x over previous
import jax
import jax.numpy as jnp
from jax import lax
from jax.experimental import pallas as pl
from jax.experimental.pallas import tpu as pltpu
from jax.experimental.pallas import tpu_sc as plsc

N_NODES = 10000
N_PAD = 10240
N_EDGES = 320000
D = 128

NC = 2
NS = 16
NW = NC * NS
EDGES_PER_W = N_EDGES // NW
CB = 40
CHUNKS = EDGES_PER_W // CB
ROWS_PER_S = N_PAD // NS


def _prep_h_body(h_ref, wf_ref, wb_ref, hf_ref, hb_ref):
    h = h_ref[...]
    hf_ref[...] = jnp.dot(h, wf_ref[...], preferred_element_type=jnp.float32)
    hb_ref[...] = jnp.dot(h, wb_ref[...], preferred_element_type=jnp.float32)


def _prep_e_body(e_ref, wf_ref, wb_ref, bf_ref, bb_ref, ef_ref, eb_ref):
    e = e_ref[...]
    ef_ref[...] = jnp.dot(e, wf_ref[...], preferred_element_type=jnp.float32) + bf_ref[...]
    eb_ref[...] = jnp.dot(e, wb_ref[...], preferred_element_type=jnp.float32) + bb_ref[...]


def _sc_body(hf_hbm, hb_hbm, ef_hbm, eb_hbm, heads_hbm, tails_hbm,
             zacc_hbm, zcnt_hbm, ones_hbm,
             acc_out, cnt_out,
             acc_sh, cnt_sh,
             hidx, tidx, hbuf, bbuf, efbuf, ebbuf, ones_v,
             gsem, ssem):
    cid = lax.axis_index("c")
    sid = lax.axis_index("s")
    wid = cid * NS + sid

    pltpu.sync_copy(zacc_hbm.at[pl.ds(sid * ROWS_PER_S, ROWS_PER_S)],
                    acc_sh.at[pl.ds(sid * ROWS_PER_S, ROWS_PER_S)])
    pltpu.sync_copy(zcnt_hbm.at[pl.ds(sid * ROWS_PER_S, ROWS_PER_S)],
                    cnt_sh.at[pl.ds(sid * ROWS_PER_S, ROWS_PER_S)])
    pltpu.sync_copy(ones_hbm, ones_v)
    plsc.subcore_barrier()

    def step(i, carry):
        base = wid * EDGES_PER_W + i * CB
        pltpu.sync_copy(heads_hbm.at[pl.ds(base, CB)], hidx)
        pltpu.sync_copy(tails_hbm.at[pl.ds(base, CB)], tidx)
        pltpu.sync_copy(hf_hbm.at[hidx], hbuf)
        pltpu.sync_copy(hb_hbm.at[tidx], bbuf)
        pltpu.sync_copy(ef_hbm.at[pl.ds(base, CB)], efbuf)
        pltpu.sync_copy(eb_hbm.at[pl.ds(base, CB)], ebbuf)
        pltpu.sync_copy(hbuf, acc_sh.at[tidx], add=True)
        pltpu.sync_copy(efbuf, acc_sh.at[tidx], add=True)
        pltpu.sync_copy(bbuf, acc_sh.at[hidx], add=True)
        pltpu.sync_copy(ebbuf, acc_sh.at[hidx], add=True)
        pltpu.sync_copy(ones_v, cnt_sh.at[tidx], add=True)
        pltpu.sync_copy(ones_v, cnt_sh.at[hidx], add=True)
        return carry

    lax.fori_loop(0, CHUNKS, step, 0)
    plsc.subcore_barrier()

    pltpu.sync_copy(acc_sh.at[pl.ds(sid * ROWS_PER_S, ROWS_PER_S)],
                    acc_out.at[cid, pl.ds(sid * ROWS_PER_S, ROWS_PER_S)])
    pltpu.sync_copy(cnt_sh.at[pl.ds(sid * ROWS_PER_S, ROWS_PER_S)],
                    cnt_out.at[cid, pl.ds(sid * ROWS_PER_S, ROWS_PER_S)])


def _final_body(acc_ref, cnt_ref, h_ref, gamma_ref, beta_ref, out_ref):
    acc = acc_ref[0] + acc_ref[1]
    n = cnt_ref[0] + cnt_ref[1]
    agg = acc / (n + 1e-07)
    x = jnp.where(agg >= 0, agg, 0.01 * agg) + h_ref[...]
    mean = jnp.mean(x, axis=-1, keepdims=True)
    var = jnp.mean(jnp.square(x - mean), axis=-1, keepdims=True)
    out_ref[...] = ((x - mean) / jnp.sqrt(var + 1e-5)) * gamma_ref[...] + beta_ref[...]


@jax.jit
def _run(H, E, ht, W_fwd, b_fwd, W_back, b_back, gamma, beta):
    heads = ht[:, 0].astype(jnp.int32)
    tails = ht[:, 1].astype(jnp.int32)

    w1f, w2f = W_fwd[:D], W_fwd[D:]
    w1b, w2b = W_back[:D], W_back[D:]

    hf, hb = pl.pallas_call(
        _prep_h_body,
        out_shape=(jax.ShapeDtypeStruct((N_NODES, D), jnp.float32),
                   jax.ShapeDtypeStruct((N_NODES, D), jnp.float32)),
    )(H, w1f, w1b)

    EB = 2560
    ef, eb = pl.pallas_call(
        _prep_e_body,
        grid=(N_EDGES // EB,),
        in_specs=[
            pl.BlockSpec((EB, D), lambda i: (i, 0)),
            pl.BlockSpec((D, D), lambda i: (0, 0)),
            pl.BlockSpec((D, D), lambda i: (0, 0)),
            pl.BlockSpec((D,), lambda i: (0,)),
            pl.BlockSpec((D,), lambda i: (0,)),
        ],
        out_specs=(pl.BlockSpec((EB, D), lambda i: (i, 0)),
                   pl.BlockSpec((EB, D), lambda i: (i, 0))),
        out_shape=(jax.ShapeDtypeStruct((N_EDGES, D), jnp.float32),
                   jax.ShapeDtypeStruct((N_EDGES, D), jnp.float32)),
    )(E, w2f, w2b, b_fwd, b_back)

    zacc = jnp.zeros((N_PAD, D), jnp.float32)
    zcnt = jnp.zeros((N_PAD,), jnp.float32)
    ones = jnp.ones((CB,), jnp.float32)

    sc = pl.kernel(
        _sc_body,
        out_type=(jax.ShapeDtypeStruct((NC, N_PAD, D), jnp.float32),
                  jax.ShapeDtypeStruct((NC, N_PAD), jnp.float32)),
        mesh=plsc.VectorSubcoreMesh(core_axis_name="c", subcore_axis_name="s"),
        scratch_types=[
            pltpu.VMEM_SHARED((N_PAD, D), jnp.float32),
            pltpu.VMEM_SHARED((N_PAD,), jnp.float32),
            pltpu.VMEM((CB,), jnp.int32),
            pltpu.VMEM((CB,), jnp.int32),
            pltpu.VMEM((CB, D), jnp.float32),
            pltpu.VMEM((CB, D), jnp.float32),
            pltpu.VMEM((CB, D), jnp.float32),
            pltpu.VMEM((CB, D), jnp.float32),
            pltpu.VMEM((CB,), jnp.float32),
            pltpu.SemaphoreType.DMA,
            pltpu.SemaphoreType.DMA,
        ],
    )
    accs, cnts = sc(hf, hb, ef, eb, heads, tails, zacc, zcnt, ones)

    RB = 2048
    cnts2 = cnts.reshape(NC, N_PAD, 1)
    hpad = jnp.pad(H, ((0, N_PAD - N_NODES), (0, 0)))
    out = pl.pallas_call(
        _final_body,
        grid=(N_PAD // RB,),
        in_specs=[
            pl.BlockSpec((NC, RB, D), lambda i: (0, i, 0)),
            pl.BlockSpec((NC, RB, 1), lambda i: (0, i, 0)),
            pl.BlockSpec((RB, D), lambda i: (i, 0)),
            pl.BlockSpec((D,), lambda i: (0,)),
            pl.BlockSpec((D,), lambda i: (0,)),
        ],
        out_specs=pl.BlockSpec((RB, D), lambda i: (i, 0)),
        out_shape=jax.ShapeDtypeStruct((N_PAD, D), jnp.float32),
    )(accs, cnts2, hpad, gamma, beta)
    return out[:N_NODES]


def kernel(H, E, ht, W_fwd, b_fwd, W_back, b_back, gamma, beta):
    return _run(H, E, ht, W_fwd, b_fwd, W_back, b_back, gamma, beta)


# speedup vs baseline: 7.0456x; 2.3382x over previous
import jax
import jax.numpy as jnp
from jax import lax
from jax.experimental import pallas as pl
from jax.experimental.pallas import tpu as pltpu
from jax.experimental.pallas import tpu_sc as plsc

N_NODES = 10000
N_PAD = 10240
N_EDGES = 320000
D = 128

NC = 2
NS = 16
NW = NC * NS
EDGES_PER_W = N_EDGES // NW
CB = 40
CHUNKS = EDGES_PER_W // CB
ROWS_PER_S = N_PAD // NS


def _prep_h_body(h_ref, wf_ref, wb_ref, hf_ref, hb_ref):
    h = h_ref[...]
    hf_ref[...] = jnp.dot(h, wf_ref[...], preferred_element_type=jnp.float32)
    hb_ref[...] = jnp.dot(h, wb_ref[...], preferred_element_type=jnp.float32)


def _prep_e_body(e_ref, wf_ref, wb_ref, bf_ref, bb_ref, ef_ref, eb_ref):
    e = e_ref[...]
    ef_ref[...] = jnp.dot(e, wf_ref[...], preferred_element_type=jnp.float32) + bf_ref[...]
    eb_ref[...] = jnp.dot(e, wb_ref[...], preferred_element_type=jnp.float32) + bb_ref[...]


def _sc_body(hf_hbm, hb_hbm, ef_hbm, eb_hbm, heads_hbm, tails_hbm,
             zacc_hbm, zcnt_hbm, ones_hbm,
             acc_out, cnt_out,
             acc_sh, cnt_sh,
             hidx0, tidx0, hidx1, tidx1,
             hbuf0, bbuf0, efbuf0, ebbuf0,
             hbuf1, bbuf1, efbuf1, ebbuf1,
             ones_v, isem, gsem, ssem):
    cid = lax.axis_index("c")
    sid = lax.axis_index("s")
    wid = cid * NS + sid

    hidx = (hidx0, hidx1)
    tidx = (tidx0, tidx1)
    bufs = ((hbuf0, bbuf0, efbuf0, ebbuf0), (hbuf1, bbuf1, efbuf1, ebbuf1))

    pltpu.sync_copy(zacc_hbm.at[pl.ds(sid * ROWS_PER_S, ROWS_PER_S)],
                    acc_sh.at[pl.ds(sid * ROWS_PER_S, ROWS_PER_S)])
    pltpu.sync_copy(zcnt_hbm.at[pl.ds(sid * ROWS_PER_S, ROWS_PER_S)],
                    cnt_sh.at[pl.ds(sid * ROWS_PER_S, ROWS_PER_S)])
    pltpu.sync_copy(ones_hbm, ones_v)
    plsc.subcore_barrier()

    def base_of(i):
        return wid * EDGES_PER_W + i * CB

    def issue_idx(p, i):
        b = base_of(i)
        pltpu.async_copy(heads_hbm.at[pl.ds(b, CB)], hidx[p], isem)
        pltpu.async_copy(tails_hbm.at[pl.ds(b, CB)], tidx[p], isem)

    def wait_idx(p):
        pltpu.make_async_copy(heads_hbm.at[pl.ds(0, CB)], hidx[p], isem).wait()
        pltpu.make_async_copy(tails_hbm.at[pl.ds(0, CB)], tidx[p], isem).wait()

    def issue_loads(p, i):
        b = base_of(i)
        pltpu.async_copy(hf_hbm.at[hidx[p]], bufs[p][0], gsem)
        pltpu.async_copy(hb_hbm.at[tidx[p]], bufs[p][1], gsem)
        pltpu.async_copy(ef_hbm.at[pl.ds(b, CB)], bufs[p][2], gsem)
        pltpu.async_copy(eb_hbm.at[pl.ds(b, CB)], bufs[p][3], gsem)

    def wait_loads(p):
        for k in range(4):
            pltpu.make_async_copy(ef_hbm.at[pl.ds(0, CB)], bufs[p][k], gsem).wait()

    def issue_scatters(p):
        pltpu.async_copy(bufs[p][0], acc_sh.at[tidx[p]], ssem, add=True)
        pltpu.async_copy(bufs[p][2], acc_sh.at[tidx[p]], ssem, add=True)
        pltpu.async_copy(bufs[p][1], acc_sh.at[hidx[p]], ssem, add=True)
        pltpu.async_copy(bufs[p][3], acc_sh.at[hidx[p]], ssem, add=True)
        pltpu.async_copy(ones_v, cnt_sh.at[tidx[p]], ssem, add=True)
        pltpu.async_copy(ones_v, cnt_sh.at[hidx[p]], ssem, add=True)

    def wait_scatters(p):
        for k in range(4):
            pltpu.make_async_copy(bufs[p][k], acc_sh.at[tidx[p]], ssem).wait()
        pltpu.make_async_copy(ones_v, cnt_sh.at[tidx[p]], ssem).wait()
        pltpu.make_async_copy(ones_v, cnt_sh.at[hidx[p]], ssem).wait()

    def chunk_step(p, i):
        wait_scatters(p ^ 1)
        issue_idx(p ^ 1, i + 1)
        wait_loads(p)
        issue_scatters(p)
        wait_idx(p ^ 1)
        issue_loads(p ^ 1, i + 1)

    issue_idx(0, 0)
    wait_idx(0)
    issue_loads(0, 0)
    issue_idx(1, 1)
    wait_loads(0)
    issue_scatters(0)
    wait_idx(1)
    issue_loads(1, 1)

    def loop_body(g, carry):
        chunk_step(1, 2 * g + 1)
        chunk_step(0, 2 * g + 2)
        return carry

    lax.fori_loop(0, (CHUNKS - 2) // 2, loop_body, 0)

    wait_scatters(0)
    wait_loads(1)
    issue_scatters(1)
    wait_scatters(1)
    plsc.subcore_barrier()

    pltpu.sync_copy(acc_sh.at[pl.ds(sid * ROWS_PER_S, ROWS_PER_S)],
                    acc_out.at[cid, pl.ds(sid * ROWS_PER_S, ROWS_PER_S)])
    pltpu.sync_copy(cnt_sh.at[pl.ds(sid * ROWS_PER_S, ROWS_PER_S)],
                    cnt_out.at[cid, pl.ds(sid * ROWS_PER_S, ROWS_PER_S)])


def _final_body(acc_ref, cnt_ref, h_ref, gamma_ref, beta_ref, out_ref):
    acc = acc_ref[0] + acc_ref[1]
    n = cnt_ref[0] + cnt_ref[1]
    agg = acc / (n + 1e-07)
    x = jnp.where(agg >= 0, agg, 0.01 * agg) + h_ref[...]
    mean = jnp.mean(x, axis=-1, keepdims=True)
    var = jnp.mean(jnp.square(x - mean), axis=-1, keepdims=True)
    out_ref[...] = ((x - mean) / jnp.sqrt(var + 1e-5)) * gamma_ref[...] + beta_ref[...]


@jax.jit
def _run(H, E, ht, W_fwd, b_fwd, W_back, b_back, gamma, beta):
    heads = ht[:, 0].astype(jnp.int32)
    tails = ht[:, 1].astype(jnp.int32)

    w1f, w2f = W_fwd[:D], W_fwd[D:]
    w1b, w2b = W_back[:D], W_back[D:]

    hf, hb = pl.pallas_call(
        _prep_h_body,
        out_shape=(jax.ShapeDtypeStruct((N_NODES, D), jnp.float32),
                   jax.ShapeDtypeStruct((N_NODES, D), jnp.float32)),
    )(H, w1f, w1b)

    EB = 2560
    ef, eb = pl.pallas_call(
        _prep_e_body,
        grid=(N_EDGES // EB,),
        in_specs=[
            pl.BlockSpec((EB, D), lambda i: (i, 0)),
            pl.BlockSpec((D, D), lambda i: (0, 0)),
            pl.BlockSpec((D, D), lambda i: (0, 0)),
            pl.BlockSpec((D,), lambda i: (0,)),
            pl.BlockSpec((D,), lambda i: (0,)),
        ],
        out_specs=(pl.BlockSpec((EB, D), lambda i: (i, 0)),
                   pl.BlockSpec((EB, D), lambda i: (i, 0))),
        out_shape=(jax.ShapeDtypeStruct((N_EDGES, D), jnp.float32),
                   jax.ShapeDtypeStruct((N_EDGES, D), jnp.float32)),
    )(E, w2f, w2b, b_fwd, b_back)

    zacc = jnp.zeros((N_PAD, D), jnp.float32)
    zcnt = jnp.zeros((N_PAD,), jnp.float32)
    ones = jnp.ones((CB,), jnp.float32)

    sc = pl.kernel(
        _sc_body,
        out_type=(jax.ShapeDtypeStruct((NC, N_PAD, D), jnp.float32),
                  jax.ShapeDtypeStruct((NC, N_PAD), jnp.float32)),
        mesh=plsc.VectorSubcoreMesh(core_axis_name="c", subcore_axis_name="s"),
        scratch_types=(
            [pltpu.VMEM_SHARED((N_PAD, D), jnp.float32),
             pltpu.VMEM_SHARED((N_PAD,), jnp.float32)]
            + [pltpu.VMEM((CB,), jnp.int32)] * 4
            + [pltpu.VMEM((CB, D), jnp.float32)] * 8
            + [pltpu.VMEM((CB,), jnp.float32)]
            + [pltpu.SemaphoreType.DMA] * 3
        ),
    )
    accs, cnts = sc(hf, hb, ef, eb, heads, tails, zacc, zcnt, ones)

    RB = 2048
    cnts2 = cnts.reshape(NC, N_PAD, 1)
    hpad = jnp.pad(H, ((0, N_PAD - N_NODES), (0, 0)))
    out = pl.pallas_call(
        _final_body,
        grid=(N_PAD // RB,),
        in_specs=[
            pl.BlockSpec((NC, RB, D), lambda i: (0, i, 0)),
            pl.BlockSpec((NC, RB, 1), lambda i: (0, i, 0)),
            pl.BlockSpec((RB, D), lambda i: (i, 0)),
            pl.BlockSpec((D,), lambda i: (0,)),
            pl.BlockSpec((D,), lambda i: (0,)),
        ],
        out_specs=pl.BlockSpec((RB, D), lambda i: (i, 0)),
        out_shape=jax.ShapeDtypeStruct((N_PAD, D), jnp.float32),
    )(accs, cnts2, hpad, gamma, beta)
    return out[:N_NODES]


def kernel(H, E, ht, W_fwd, b_fwd, W_back, b_back, gamma, beta):
    return _run(H, E, ht, W_fwd, b_fwd, W_back, b_back, gamma, beta)


# speedup vs baseline: 8.0907x; 1.1483x over previous
import jax
import jax.numpy as jnp
from jax import lax
from jax.experimental import pallas as pl
from jax.experimental.pallas import tpu as pltpu
from jax.experimental.pallas import tpu_sc as plsc

N_NODES = 10000
N_PAD = 10240
N_EDGES = 320000
D = 128

NC = 2
NS = 16
NW = NC * NS
EDGES_PER_W = N_EDGES // NW
CB = 80
CHUNKS = EDGES_PER_W // CB
ROWS_PER_S = N_PAD // NS


def _prep_h_body(h_ref, wf_ref, wb_ref, hf_ref, hb_ref):
    h = h_ref[...]
    hf_ref[...] = jnp.dot(h, wf_ref[...], preferred_element_type=jnp.float32)
    hb_ref[...] = jnp.dot(h, wb_ref[...], preferred_element_type=jnp.float32)


def _prep_e_body(e_ref, wf_ref, wb_ref, bf_ref, bb_ref, ef_ref, eb_ref):
    e = e_ref[...]
    ef_ref[...] = jnp.dot(e, wf_ref[...], preferred_element_type=jnp.float32) + bf_ref[...]
    eb_ref[...] = jnp.dot(e, wb_ref[...], preferred_element_type=jnp.float32) + bb_ref[...]


def _run_pipeline(chunk_fns):
    issue_idx, wait_idx, issue_loads, wait_loads, issue_scatters, wait_scatters = chunk_fns

    def chunk_step(p, i):
        wait_scatters(p ^ 1)
        issue_idx(p ^ 1, i + 1)
        wait_loads(p)
        issue_scatters(p)
        wait_idx(p ^ 1)
        issue_loads(p ^ 1, i + 1)

    issue_idx(0, 0)
    wait_idx(0)
    issue_loads(0, 0)
    issue_idx(1, 1)
    wait_loads(0)
    issue_scatters(0)
    wait_idx(1)
    issue_loads(1, 1)

    def loop_body(g, carry):
        chunk_step(1, 2 * g + 1)
        chunk_step(0, 2 * g + 2)
        return carry

    K = (CHUNKS - 2) // 2
    lax.fori_loop(0, K, loop_body, 0)
    if CHUNKS % 2:
        chunk_step(1, 2 * K + 1)
        last_p = 0
    else:
        last_p = 1

    wait_scatters(last_p ^ 1)
    wait_loads(last_p)
    issue_scatters(last_p)
    wait_scatters(last_p)


def _sc_h_body(hf_hbm, hb_hbm, heads_hbm, tails_hbm,
               zacc_hbm, zcnt_hbm, ones_hbm,
               acc_out, cnt_out,
               acc_sh, cnt_sh,
               hidx0, tidx0, hidx1, tidx1,
               hbuf0, bbuf0, hbuf1, bbuf1,
               ones_v, isem, gsem, ssem):
    cid = lax.axis_index("c")
    sid = lax.axis_index("s")
    wid = cid * NS + sid

    hidx = (hidx0, hidx1)
    tidx = (tidx0, tidx1)
    bufs = ((hbuf0, bbuf0), (hbuf1, bbuf1))

    pltpu.sync_copy(zacc_hbm.at[pl.ds(sid * ROWS_PER_S, ROWS_PER_S)],
                    acc_sh.at[pl.ds(sid * ROWS_PER_S, ROWS_PER_S)])
    pltpu.sync_copy(zcnt_hbm.at[pl.ds(sid * ROWS_PER_S, ROWS_PER_S)],
                    cnt_sh.at[pl.ds(sid * ROWS_PER_S, ROWS_PER_S)])
    pltpu.sync_copy(ones_hbm, ones_v)
    plsc.subcore_barrier()

    def base_of(i):
        return wid * EDGES_PER_W + i * CB

    def issue_idx(p, i):
        b = base_of(i)
        pltpu.async_copy(heads_hbm.at[pl.ds(b, CB)], hidx[p], isem)
        pltpu.async_copy(tails_hbm.at[pl.ds(b, CB)], tidx[p], isem)

    def wait_idx(p):
        pltpu.make_async_copy(heads_hbm.at[pl.ds(0, CB)], hidx[p], isem).wait()
        pltpu.make_async_copy(tails_hbm.at[pl.ds(0, CB)], tidx[p], isem).wait()

    def issue_loads(p, i):
        pltpu.async_copy(hf_hbm.at[hidx[p]], bufs[p][0], gsem)
        pltpu.async_copy(hb_hbm.at[tidx[p]], bufs[p][1], gsem)

    def wait_loads(p):
        for k in range(2):
            pltpu.make_async_copy(hf_hbm.at[pl.ds(0, CB)], bufs[p][k], gsem).wait()

    def issue_scatters(p):
        pltpu.async_copy(bufs[p][0], acc_sh.at[tidx[p]], ssem, add=True)
        pltpu.async_copy(bufs[p][1], acc_sh.at[hidx[p]], ssem, add=True)
        pltpu.async_copy(ones_v, cnt_sh.at[tidx[p]], ssem, add=True)
        pltpu.async_copy(ones_v, cnt_sh.at[hidx[p]], ssem, add=True)

    def wait_scatters(p):
        for k in range(2):
            pltpu.make_async_copy(bufs[p][k], acc_sh.at[tidx[p]], ssem).wait()
        pltpu.make_async_copy(ones_v, cnt_sh.at[tidx[p]], ssem).wait()
        pltpu.make_async_copy(ones_v, cnt_sh.at[hidx[p]], ssem).wait()

    _run_pipeline((issue_idx, wait_idx, issue_loads, wait_loads,
                   issue_scatters, wait_scatters))
    plsc.subcore_barrier()

    pltpu.sync_copy(acc_sh.at[pl.ds(sid * ROWS_PER_S, ROWS_PER_S)],
                    acc_out.at[cid, pl.ds(sid * ROWS_PER_S, ROWS_PER_S)])
    pltpu.sync_copy(cnt_sh.at[pl.ds(sid * ROWS_PER_S, ROWS_PER_S)],
                    cnt_out.at[cid, pl.ds(sid * ROWS_PER_S, ROWS_PER_S)])


def _sc_e_body(ef_hbm, eb_hbm, heads_hbm, tails_hbm,
               zacc_hbm,
               acc_out,
               acc_sh,
               hidx0, tidx0, hidx1, tidx1,
               fbuf0, bbuf0, fbuf1, bbuf1,
               isem, gsem, ssem):
    cid = lax.axis_index("c")
    sid = lax.axis_index("s")
    wid = cid * NS + sid

    hidx = (hidx0, hidx1)
    tidx = (tidx0, tidx1)
    bufs = ((fbuf0, bbuf0), (fbuf1, bbuf1))

    pltpu.sync_copy(zacc_hbm.at[pl.ds(sid * ROWS_PER_S, ROWS_PER_S)],
                    acc_sh.at[pl.ds(sid * ROWS_PER_S, ROWS_PER_S)])
    plsc.subcore_barrier()

    def base_of(i):
        return wid * EDGES_PER_W + i * CB

    def issue_idx(p, i):
        b = base_of(i)
        pltpu.async_copy(heads_hbm.at[pl.ds(b, CB)], hidx[p], isem)
        pltpu.async_copy(tails_hbm.at[pl.ds(b, CB)], tidx[p], isem)

    def wait_idx(p):
        pltpu.make_async_copy(heads_hbm.at[pl.ds(0, CB)], hidx[p], isem).wait()
        pltpu.make_async_copy(tails_hbm.at[pl.ds(0, CB)], tidx[p], isem).wait()

    def issue_loads(p, i):
        b = base_of(i)
        pltpu.async_copy(ef_hbm.at[pl.ds(b, CB)], bufs[p][0], gsem)
        pltpu.async_copy(eb_hbm.at[pl.ds(b, CB)], bufs[p][1], gsem)

    def wait_loads(p):
        for k in range(2):
            pltpu.make_async_copy(ef_hbm.at[pl.ds(0, CB)], bufs[p][k], gsem).wait()

    def issue_scatters(p):
        pltpu.async_copy(bufs[p][0], acc_sh.at[tidx[p]], ssem, add=True)
        pltpu.async_copy(bufs[p][1], acc_sh.at[hidx[p]], ssem, add=True)

    def wait_scatters(p):
        for k in range(2):
            pltpu.make_async_copy(bufs[p][k], acc_sh.at[tidx[p]], ssem).wait()

    _run_pipeline((issue_idx, wait_idx, issue_loads, wait_loads,
                   issue_scatters, wait_scatters))
    plsc.subcore_barrier()

    pltpu.sync_copy(acc_sh.at[pl.ds(sid * ROWS_PER_S, ROWS_PER_S)],
                    acc_out.at[cid, pl.ds(sid * ROWS_PER_S, ROWS_PER_S)])


def _final_body(acch_ref, acce_ref, cnt_ref, h_ref, gamma_ref, beta_ref, out_ref):
    acc = acch_ref[0] + acch_ref[1] + acce_ref[0] + acce_ref[1]
    n = cnt_ref[0] + cnt_ref[1]
    agg = acc / (n + 1e-07)
    x = jnp.where(agg >= 0, agg, 0.01 * agg) + h_ref[...]
    mean = jnp.mean(x, axis=-1, keepdims=True)
    var = jnp.mean(jnp.square(x - mean), axis=-1, keepdims=True)
    out_ref[...] = ((x - mean) / jnp.sqrt(var + 1e-5)) * gamma_ref[...] + beta_ref[...]


@jax.jit
def _run(H, E, ht, W_fwd, b_fwd, W_back, b_back, gamma, beta):
    heads = ht[:, 0].astype(jnp.int32)
    tails = ht[:, 1].astype(jnp.int32)

    w1f, w2f = W_fwd[:D], W_fwd[D:]
    w1b, w2b = W_back[:D], W_back[D:]

    hf, hb = pl.pallas_call(
        _prep_h_body,
        out_shape=(jax.ShapeDtypeStruct((N_NODES, D), jnp.float32),
                   jax.ShapeDtypeStruct((N_NODES, D), jnp.float32)),
    )(H, w1f, w1b)

    zacc = jnp.zeros((N_PAD, D), jnp.float32)
    zcnt = jnp.zeros((N_PAD,), jnp.float32)
    ones = jnp.ones((CB,), jnp.float32)

    sc_h = pl.kernel(
        _sc_h_body,
        out_type=(jax.ShapeDtypeStruct((NC, N_PAD, D), jnp.float32),
                  jax.ShapeDtypeStruct((NC, N_PAD), jnp.float32)),
        mesh=plsc.VectorSubcoreMesh(core_axis_name="c", subcore_axis_name="s"),
        scratch_types=(
            [pltpu.VMEM_SHARED((N_PAD, D), jnp.float32),
             pltpu.VMEM_SHARED((N_PAD,), jnp.float32)]
            + [pltpu.VMEM((CB,), jnp.int32)] * 4
            + [pltpu.VMEM((CB, D), jnp.float32)] * 4
            + [pltpu.VMEM((CB,), jnp.float32)]
            + [pltpu.SemaphoreType.DMA] * 3
        ),
    )
    acch, cnts = sc_h(hf, hb, heads, tails, zacc, zcnt, ones)

    EB = 2560
    ef, eb = pl.pallas_call(
        _prep_e_body,
        grid=(N_EDGES // EB,),
        in_specs=[
            pl.BlockSpec((EB, D), lambda i: (i, 0)),
            pl.BlockSpec((D, D), lambda i: (0, 0)),
            pl.BlockSpec((D, D), lambda i: (0, 0)),
            pl.BlockSpec((D,), lambda i: (0,)),
            pl.BlockSpec((D,), lambda i: (0,)),
        ],
        out_specs=(pl.BlockSpec((EB, D), lambda i: (i, 0)),
                   pl.BlockSpec((EB, D), lambda i: (i, 0))),
        out_shape=(jax.ShapeDtypeStruct((N_EDGES, D), jnp.float32),
                   jax.ShapeDtypeStruct((N_EDGES, D), jnp.float32)),
    )(E, w2f, w2b, b_fwd, b_back)

    sc_e = pl.kernel(
        _sc_e_body,
        out_type=jax.ShapeDtypeStruct((NC, N_PAD, D), jnp.float32),
        mesh=plsc.VectorSubcoreMesh(core_axis_name="c", subcore_axis_name="s"),
        scratch_types=(
            [pltpu.VMEM_SHARED((N_PAD, D), jnp.float32)]
            + [pltpu.VMEM((CB,), jnp.int32)] * 4
            + [pltpu.VMEM((CB, D), jnp.float32)] * 4
            + [pltpu.SemaphoreType.DMA] * 3
        ),
    )
    acce = sc_e(ef, eb, heads, tails, zacc)

    RB = 2048
    cnts2 = cnts.reshape(NC, N_PAD, 1)
    hpad = jnp.pad(H, ((0, N_PAD - N_NODES), (0, 0)))
    out = pl.pallas_call(
        _final_body,
        grid=(N_PAD // RB,),
        in_specs=[
            pl.BlockSpec((NC, RB, D), lambda i: (0, i, 0)),
            pl.BlockSpec((NC, RB, D), lambda i: (0, i, 0)),
            pl.BlockSpec((NC, RB, 1), lambda i: (0, i, 0)),
            pl.BlockSpec((RB, D), lambda i: (i, 0)),
            pl.BlockSpec((D,), lambda i: (0,)),
            pl.BlockSpec((D,), lambda i: (0,)),
        ],
        out_specs=pl.BlockSpec((RB, D), lambda i: (i, 0)),
        out_shape=jax.ShapeDtypeStruct((N_PAD, D), jnp.float32),
    )(acch, acce, cnts2, hpad, gamma, beta)
    return out[:N_NODES]


def kernel(H, E, ht, W_fwd, b_fwd, W_back, b_back, gamma, beta):
    return _run(H, E, ht, W_fwd, b_fwd, W_back, b_back, gamma, beta)


# speedup vs baseline: 8.4443x; 1.0437x over previous
import jax
import jax.numpy as jnp
from jax import lax
from jax.experimental import pallas as pl
from jax.experimental.pallas import tpu as pltpu
from jax.experimental.pallas import tpu_sc as plsc

N_NODES = 10000
N_PAD = 10240
N_EDGES = 320000
D = 128

NC = 2
NS = 16
NW = NC * NS

CB = 80
EDGES_PER_W = N_EDGES // NW
CHUNKS_H = EDGES_PER_W // CB
EDGES_PER_T = N_EDGES // NS
CHUNKS_E = EDGES_PER_T // CB
ROWS_PER_S = N_PAD // NS


def _prep_h_body(h_ref, wf_ref, wb_ref, hf_ref, hb_ref):
    h = h_ref[...]
    hf_ref[...] = jnp.dot(h, wf_ref[...], preferred_element_type=jnp.float32)
    hb_ref[...] = jnp.dot(h, wb_ref[...], preferred_element_type=jnp.float32)


def _run_pipeline(chunks, chunk_fns):
    issue_idx, wait_idx, issue_loads, wait_loads, issue_scatters, wait_scatters = chunk_fns

    def chunk_step(p, i):
        wait_scatters(p ^ 1)
        issue_idx(p ^ 1, i + 1)
        wait_loads(p)
        issue_scatters(p)
        wait_idx(p ^ 1)
        issue_loads(p ^ 1, i + 1)

    issue_idx(0, 0)
    wait_idx(0)
    issue_loads(0, 0)
    issue_idx(1, 1)
    wait_loads(0)
    issue_scatters(0)
    wait_idx(1)
    issue_loads(1, 1)

    def loop_body(g, carry):
        chunk_step(1, 2 * g + 1)
        chunk_step(0, 2 * g + 2)
        return carry

    K = (chunks - 2) // 2
    lax.fori_loop(0, K, loop_body, 0)
    if chunks % 2:
        chunk_step(1, 2 * K + 1)
        last_p = 0
    else:
        last_p = 1

    wait_scatters(last_p ^ 1)
    wait_loads(last_p)
    issue_scatters(last_p)
    wait_scatters(last_p)


def _sc_h_body(hf_hbm, hb_hbm, heads_hbm, tails_hbm,
               zacc_hbm,
               acc_out,
               acc_sh,
               hidx0, tidx0, hidx1, tidx1,
               hbuf0, bbuf0, hbuf1, bbuf1,
               isem, gsem, ssem):
    cid = lax.axis_index("c")
    sid = lax.axis_index("s")
    wid = cid * NS + sid

    hidx = (hidx0, hidx1)
    tidx = (tidx0, tidx1)
    bufs = ((hbuf0, bbuf0), (hbuf1, bbuf1))

    pltpu.sync_copy(zacc_hbm.at[pl.ds(sid * ROWS_PER_S, ROWS_PER_S)],
                    acc_sh.at[pl.ds(sid * ROWS_PER_S, ROWS_PER_S)])
    plsc.subcore_barrier()

    def base_of(i):
        return wid * EDGES_PER_W + i * CB

    def issue_idx(p, i):
        b = base_of(i)
        pltpu.async_copy(heads_hbm.at[pl.ds(b, CB)], hidx[p], isem)
        pltpu.async_copy(tails_hbm.at[pl.ds(b, CB)], tidx[p], isem)

    def wait_idx(p):
        pltpu.make_async_copy(heads_hbm.at[pl.ds(0, CB)], hidx[p], isem).wait()
        pltpu.make_async_copy(tails_hbm.at[pl.ds(0, CB)], tidx[p], isem).wait()

    def issue_loads(p, i):
        pltpu.async_copy(hf_hbm.at[hidx[p]], bufs[p][0], gsem)
        pltpu.async_copy(hb_hbm.at[tidx[p]], bufs[p][1], gsem)

    def wait_loads(p):
        for k in range(2):
            pltpu.make_async_copy(hf_hbm.at[pl.ds(0, CB)], bufs[p][k], gsem).wait()

    def issue_scatters(p):
        pltpu.async_copy(bufs[p][0], acc_sh.at[tidx[p]], ssem, add=True)
        pltpu.async_copy(bufs[p][1], acc_sh.at[hidx[p]], ssem, add=True)

    def wait_scatters(p):
        for k in range(2):
            pltpu.make_async_copy(bufs[p][k], acc_sh.at[tidx[p]], ssem).wait()

    _run_pipeline(CHUNKS_H, (issue_idx, wait_idx, issue_loads, wait_loads,
                             issue_scatters, wait_scatters))
    plsc.subcore_barrier()

    pltpu.sync_copy(acc_sh.at[pl.ds(sid * ROWS_PER_S, ROWS_PER_S)],
                    acc_out.at[cid, pl.ds(sid * ROWS_PER_S, ROWS_PER_S)])


def _sc_e_body(e_hbm, dst_hbm,
               zacc_hbm, zcnt_hbm, ones_hbm,
               acc_out, cnt_out,
               acc_sh, cnt_sh,
               didx0, didx1,
               ebuf0, ebuf1,
               ones_v, isem, gsem, ssem):
    cid = lax.axis_index("c")
    sid = lax.axis_index("s")

    didx = (didx0, didx1)
    ebuf = (ebuf0, ebuf1)

    pltpu.sync_copy(zacc_hbm.at[pl.ds(sid * ROWS_PER_S, ROWS_PER_S)],
                    acc_sh.at[pl.ds(sid * ROWS_PER_S, ROWS_PER_S)])
    pltpu.sync_copy(zcnt_hbm.at[pl.ds(sid * ROWS_PER_S, ROWS_PER_S)],
                    cnt_sh.at[pl.ds(sid * ROWS_PER_S, ROWS_PER_S)])
    pltpu.sync_copy(ones_hbm, ones_v)
    plsc.subcore_barrier()

    def base_of(i):
        return sid * EDGES_PER_T + i * CB

    def issue_idx(p, i):
        b = base_of(i)
        pltpu.async_copy(dst_hbm.at[pl.ds(cid * N_EDGES + b, CB)], didx[p], isem)

    def wait_idx(p):
        pltpu.make_async_copy(dst_hbm.at[pl.ds(0, CB)], didx[p], isem).wait()

    def issue_loads(p, i):
        b = base_of(i)
        pltpu.async_copy(e_hbm.at[pl.ds(b, CB)], ebuf[p], gsem)

    def wait_loads(p):
        pltpu.make_async_copy(e_hbm.at[pl.ds(0, CB)], ebuf[p], gsem).wait()

    def issue_scatters(p):
        pltpu.async_copy(ebuf[p], acc_sh.at[didx[p]], ssem, add=True)
        pltpu.async_copy(ones_v, cnt_sh.at[didx[p]], ssem, add=True)

    def wait_scatters(p):
        pltpu.make_async_copy(ebuf[p], acc_sh.at[didx[p]], ssem).wait()
        pltpu.make_async_copy(ones_v, cnt_sh.at[didx[p]], ssem).wait()

    _run_pipeline(CHUNKS_E, (issue_idx, wait_idx, issue_loads, wait_loads,
                             issue_scatters, wait_scatters))
    plsc.subcore_barrier()

    pltpu.sync_copy(acc_sh.at[pl.ds(sid * ROWS_PER_S, ROWS_PER_S)],
                    acc_out.at[cid, pl.ds(sid * ROWS_PER_S, ROWS_PER_S)])
    pltpu.sync_copy(cnt_sh.at[pl.ds(sid * ROWS_PER_S, ROWS_PER_S)],
                    cnt_out.at[cid, pl.ds(sid * ROWS_PER_S, ROWS_PER_S)])


def _final_body(acch_ref, acce_ref, cnt_ref, h_ref,
                w2f_ref, w2b_ref, bf_ref, bb_ref, gamma_ref, beta_ref, out_ref):
    cf = cnt_ref[0]
    cb = cnt_ref[1]
    acc = (acch_ref[0] + acch_ref[1]
           + jnp.dot(acce_ref[0], w2f_ref[...], preferred_element_type=jnp.float32)
           + jnp.dot(acce_ref[1], w2b_ref[...], preferred_element_type=jnp.float32)
           + cf * bf_ref[...] + cb * bb_ref[...])
    agg = acc / (cf + cb + 1e-07)
    x = jnp.where(agg >= 0, agg, 0.01 * agg) + h_ref[...]
    mean = jnp.mean(x, axis=-1, keepdims=True)
    var = jnp.mean(jnp.square(x - mean), axis=-1, keepdims=True)
    out_ref[...] = ((x - mean) / jnp.sqrt(var + 1e-5)) * gamma_ref[...] + beta_ref[...]


@jax.jit
def _run(H, E, ht, W_fwd, b_fwd, W_back, b_back, gamma, beta):
    heads = ht[:, 0].astype(jnp.int32)
    tails = ht[:, 1].astype(jnp.int32)
    dst = jnp.concatenate([tails, heads])

    w1f, w2f = W_fwd[:D], W_fwd[D:]
    w1b, w2b = W_back[:D], W_back[D:]

    hf, hb = pl.pallas_call(
        _prep_h_body,
        out_shape=(jax.ShapeDtypeStruct((N_NODES, D), jnp.float32),
                   jax.ShapeDtypeStruct((N_NODES, D), jnp.float32)),
    )(H, w1f, w1b)

    zacc = jnp.zeros((N_PAD, D), jnp.float32)
    zcnt = jnp.zeros((N_PAD,), jnp.float32)
    ones = jnp.ones((CB,), jnp.float32)

    sc_h = pl.kernel(
        _sc_h_body,
        out_type=jax.ShapeDtypeStruct((NC, N_PAD, D), jnp.float32),
        mesh=plsc.VectorSubcoreMesh(core_axis_name="c", subcore_axis_name="s"),
        scratch_types=(
            [pltpu.VMEM_SHARED((N_PAD, D), jnp.float32)]
            + [pltpu.VMEM((CB,), jnp.int32)] * 4
            + [pltpu.VMEM((CB, D), jnp.float32)] * 4
            + [pltpu.SemaphoreType.DMA] * 3
        ),
    )
    acch = sc_h(hf, hb, heads, tails, zacc)

    sc_e = pl.kernel(
        _sc_e_body,
        out_type=(jax.ShapeDtypeStruct((NC, N_PAD, D), jnp.float32),
                  jax.ShapeDtypeStruct((NC, N_PAD), jnp.float32)),
        mesh=plsc.VectorSubcoreMesh(core_axis_name="c", subcore_axis_name="s"),
        scratch_types=(
            [pltpu.VMEM_SHARED((N_PAD, D), jnp.float32),
             pltpu.VMEM_SHARED((N_PAD,), jnp.float32)]
            + [pltpu.VMEM((CB,), jnp.int32)] * 2
            + [pltpu.VMEM((CB, D), jnp.float32)] * 2
            + [pltpu.VMEM((CB,), jnp.float32)]
            + [pltpu.SemaphoreType.DMA] * 3
        ),
    )
    acce, cnts = sc_e(E, dst, zacc, zcnt, ones)

    RB = 2048
    cnts2 = cnts.reshape(NC, N_PAD, 1)
    hpad = jnp.pad(H, ((0, N_PAD - N_NODES), (0, 0)))
    out = pl.pallas_call(
        _final_body,
        grid=(N_PAD // RB,),
        in_specs=[
            pl.BlockSpec((NC, RB, D), lambda i: (0, i, 0)),
            pl.BlockSpec((NC, RB, D), lambda i: (0, i, 0)),
            pl.BlockSpec((NC, RB, 1), lambda i: (0, i, 0)),
            pl.BlockSpec((RB, D), lambda i: (i, 0)),
            pl.BlockSpec((D, D), lambda i: (0, 0)),
            pl.BlockSpec((D, D), lambda i: (0, 0)),
            pl.BlockSpec((D,), lambda i: (0,)),
            pl.BlockSpec((D,), lambda i: (0,)),
            pl.BlockSpec((D,), lambda i: (0,)),
            pl.BlockSpec((D,), lambda i: (0,)),
        ],
        out_specs=pl.BlockSpec((RB, D), lambda i: (i, 0)),
        out_shape=jax.ShapeDtypeStruct((N_PAD, D), jnp.float32),
    )(acch, acce, cnts2, hpad, w2f, w2b, b_fwd, b_back, gamma, beta)
    return out[:N_NODES]


def kernel(H, E, ht, W_fwd, b_fwd, W_back, b_back, gamma, beta):
    return _run(H, E, ht, W_fwd, b_fwd, W_back, b_back, gamma, beta)
